# baseline (speedup 1.0000x reference)
import functools

import jax
import jax.numpy as jnp
from jax import lax
from jax.experimental import pallas as pl
from jax.experimental.pallas import tpu as pltpu
from jax.experimental.pallas import tpu_sc as plsc

_NC = 2
_NS = 16
_NW = _NC * _NS

_D = 32
_R = 16384
_S = 50
_RPW = _R // _NW
_RC = 32
_C = _RC * _S


def _gather_body(table_hbm, x_hbm, out_hbm, idx_v, rows_v, sem, osem):
    wid = lax.axis_index("s") * _NC + lax.axis_index("c")
    base = wid * _RPW

    def chunk(i, carry):
        i0 = base + i * _RC
        pltpu.sync_copy(x_hbm.at[pl.ds(i0, _RC), :], idx_v)
        gh = [
            pltpu.async_copy(
                table_hbm.at[idx_v.at[r]], rows_v.at[pl.ds(r * _S, _S), :], sem
            )
            for r in range(_RC)
        ]
        for h in gh:
            h.wait()
        oh = [
            pltpu.async_copy(
                rows_v.at[pl.ds(r * _S, _S), :], out_hbm.at[i0 + r], osem
            )
            for r in range(_RC)
        ]
        for h in oh:
            h.wait()
        return carry

    lax.fori_loop(0, _RPW // _RC, chunk, 0)


@jax.jit
def _embedding_lookup(W, x):
    mesh = plsc.VectorSubcoreMesh(core_axis_name="c", subcore_axis_name="s")
    f = functools.partial(
        pl.kernel,
        mesh=mesh,
        out_type=jax.ShapeDtypeStruct((_R, _S, _D), jnp.float32),
        scratch_types=[
            pltpu.VMEM((_RC, _S), jnp.int32),
            pltpu.VMEM((_C, _D), jnp.float32),
            pltpu.SemaphoreType.DMA,
            pltpu.SemaphoreType.DMA,
        ],
        compiler_params=pltpu.CompilerParams(use_tc_tiling_on_sc=False),
    )(_gather_body)
    return f(W, x)


def kernel(x, W):
    return _embedding_lookup(W, x)
